# baseline (speedup 1.0000x reference)
import functools

import jax
import jax.numpy as jnp
from jax import lax
from jax.experimental import pallas as pl
from jax.experimental.pallas import tpu as pltpu, tpu_sc as plsc

B = 16384
DIM = 64
CH = 64
NBUF = 3
RB = 8000


def _tc_repack(table, one):
    n = table.shape[0]
    return table.reshape(n // 2, 2 * DIM) * one


@functools.lru_cache(maxsize=None)
def _build_user(num_cores, num_subcores):
    NW = num_cores * num_subcores
    b_per_w = B // NW
    UG = b_per_w // 16
    WCH = 128

    mesh = plsc.VectorSubcoreMesh(core_axis_name="c", subcore_axis_name="s")

    @functools.partial(
        pl.kernel,
        mesh=mesh,
        out_type=jax.ShapeDtypeStruct((B, DIM), jnp.float32),
        scratch_types=[
            pltpu.VMEM((b_per_w,), jnp.int32),
            pltpu.VMEM((b_per_w, DIM), jnp.float32),
            pltpu.SemaphoreType.DMA,
            pltpu.SemaphoreType.DMA,
        ],
    )
    def sc_user(u_i, utab, out_u, uidx, ubuf, gsem, wsem):
        wid = lax.axis_index("s") * num_cores + lax.axis_index("c")
        base = wid * b_per_w
        pltpu.sync_copy(u_i.at[wid], uidx)

        def issue(g, carry):
            v = uidx[pl.ds(g * 16, 16)]
            for l in range(16):
                pltpu.async_copy(utab.at[pl.ds(v[l], 1)],
                                 ubuf.at[pl.ds(g * 16 + l, 1)], gsem)
            return carry

        lax.fori_loop(0, UG, issue, 0)

        def drain(j, carry):
            pltpu.make_async_copy(utab.at[pl.ds(0, 1)],
                                  ubuf.at[pl.ds(0, 1)], gsem).wait()
            return carry

        lax.fori_loop(0, b_per_w, drain, 0)
        for c in range(b_per_w // WCH):
            pltpu.async_copy(ubuf.at[pl.ds(c * WCH, WCH)],
                             out_u.at[pl.ds(base + c * WCH, WCH)], wsem)
        for c in range(b_per_w // WCH):
            pltpu.make_async_copy(ubuf.at[pl.ds(c * WCH, WCH)],
                                  out_u.at[pl.ds(base + c * WCH, WCH)],
                                  wsem).wait()

    return sc_user, NW, b_per_w


@functools.lru_cache(maxsize=None)
def _build_items(num_cores, num_subcores):
    NW = num_cores * num_subcores
    b_per_w = B // NW
    NCH = b_per_w // CH
    G = CH // 16

    mesh = plsc.VectorSubcoreMesh(core_axis_name="c", subcore_axis_name="s")
    out_sds = jax.ShapeDtypeStruct((B, DIM), jnp.float32)

    @functools.partial(
        pl.kernel,
        mesh=mesh,
        out_type=(out_sds, out_sds),
        scratch_types=[
            pltpu.VMEM((b_per_w,), jnp.int32),
            pltpu.VMEM((b_per_w,), jnp.int32),
            pltpu.VMEM((b_per_w,), jnp.int32),
            pltpu.VMEM((b_per_w,), jnp.int32),
            [pltpu.VMEM((CH, 2 * DIM), jnp.float32) for _ in range(NBUF)],
            [pltpu.VMEM((CH, DIM), jnp.float32) for _ in range(NBUF)],
            [pltpu.SemaphoreType.DMA for _ in range(NBUF)],
            [pltpu.SemaphoreType.DMA for _ in range(NBUF)],
        ],
        compiler_params=pltpu.CompilerParams(needs_layout_passes=False),
    )
    def sc_items(p_p, p_h, n_p, n_h, itab, out_p, out_n,
                 ppv, phv, npv, nhv, pairs, rows, gsems, wsems):
        wid = lax.axis_index("s") * num_cores + lax.axis_index("c")
        base = wid * b_per_w

        pltpu.sync_copy(p_p.at[wid], ppv)
        pltpu.sync_copy(p_h.at[wid], phv)
        pltpu.sync_copy(n_p.at[wid], npv)
        pltpu.sync_copy(n_h.at[wid], nhv)

        sched = []
        for pv, hv, out in ((ppv, phv, out_p), (npv, nhv, out_n)):
            for c in range(NCH):
                sched.append((pv, hv, out, c * CH))
        total = len(sched)

        def fire(slot):
            pv, _, _, ofs = sched[slot]
            pltpu.async_copy(itab.at[pv.at[pl.ds(ofs, CH)]],
                             pairs[slot % NBUF], gsems[slot % NBUF])

        def drain_gather(slot):
            pv = sched[slot][0]
            pltpu.make_async_copy(itab.at[pv.at[pl.ds(0, CH)]],
                                  pairs[slot % NBUF], gsems[slot % NBUF]).wait()

        def extract(slot):
            _, hv, _, ofs = sched[slot]
            pbuf = pairs[slot % NBUF]
            rbuf = rows[slot % NBUF]

            def group(g, carry):
                jrow = lax.iota(jnp.int32, 16) + g * 16
                hvec = hv[pl.ds(ofs + g * 16, 16)]
                for col in range(DIM):
                    x = plsc.load_gather(pbuf, [jrow, hvec + col])
                    plsc.store_scatter(rbuf, [jrow, jnp.full((16,), col, jnp.int32)], x)
                return carry

            lax.fori_loop(0, G, group, 0)

        def start_writeback(slot):
            _, _, out, ofs = sched[slot]
            pltpu.async_copy(rows[slot % NBUF], out.at[pl.ds(base + ofs, CH)],
                             wsems[slot % NBUF])

        def drain_writeback(slot):
            _, _, out, ofs = sched[slot]
            pltpu.make_async_copy(rows[slot % NBUF], out.at[pl.ds(base + ofs, CH)],
                                  wsems[slot % NBUF]).wait()

        for s in range(min(NBUF - 1, total)):
            fire(s)
        for s in range(total):
            drain_gather(s)
            if s >= NBUF:
                drain_writeback(s - NBUF)
            extract(s)
            start_writeback(s)
            nxt = s + NBUF - 1
            if nxt < total:
                fire(nxt)
        for s in range(max(total - NBUF, 0), total):
            drain_writeback(s)

    return sc_items, NW, b_per_w


def kernel(user, pos, neg, user_table, item_table):
    info = plsc.get_sparse_core_info()
    fn_u, nw, bw = _build_user(info.num_cores, info.num_subcores)
    fn_i, _, _ = _build_items(info.num_cores, info.num_subcores)

    def split(idx):
        idx = idx.astype(jnp.int32)
        return ((idx >> 1).reshape(nw, bw),
                ((idx & 1) * DIM).reshape(nw, bw))

    u = user.astype(jnp.int32).reshape(nw, bw)
    p_p, p_h = split(pos)
    n_p, n_h = split(neg)
    one = (user[0] // jnp.int32(2**30) + 1).astype(jnp.float32)
    it2 = _tc_repack(item_table, one)
    out_u = fn_u(u, user_table)
    out_p, out_n = fn_i(p_p, p_h, n_p, n_h, it2)
    return (out_u, out_p, out_n)


# speedup vs baseline: 1.5347x; 1.5347x over previous
import functools

import jax
import jax.numpy as jnp
from jax import lax
from jax.experimental import pallas as pl
from jax.experimental.pallas import tpu as pltpu, tpu_sc as plsc

B = 16384
DIM = 64
CH = 128
NBUF = 3
NSEM = 8


@functools.lru_cache(maxsize=None)
def _build(num_cores, num_subcores):
    NW = num_cores * num_subcores
    b_per_w = B // NW
    NCH = b_per_w // CH
    G = CH // 16

    mesh = plsc.VectorSubcoreMesh(core_axis_name="c", subcore_axis_name="s")
    out_sds = jax.ShapeDtypeStruct((B, DIM), jnp.float32)

    @functools.partial(
        pl.kernel,
        mesh=mesh,
        out_type=(out_sds, out_sds, out_sds),
        scratch_types=[
            pltpu.VMEM((b_per_w,), jnp.int32),
            pltpu.VMEM((b_per_w,), jnp.int32),
            pltpu.VMEM((b_per_w,), jnp.int32),
            [pltpu.VMEM((CH, DIM), jnp.float32) for _ in range(NBUF)],
            [[pltpu.SemaphoreType.DMA for _ in range(NSEM)]
             for _ in range(NBUF)],
            [pltpu.SemaphoreType.DMA for _ in range(NBUF)],
        ],
    )
    def sc_gather3(u_i, p_i, n_i, utab, itab, out_u, out_p, out_n,
                   uidx, pidx, nidx, bufs, gsems, wsems):
        wid = lax.axis_index("s") * num_cores + lax.axis_index("c")
        base = wid * b_per_w

        pltpu.sync_copy(u_i.at[wid], uidx)
        pltpu.sync_copy(p_i.at[wid], pidx)
        pltpu.sync_copy(n_i.at[wid], nidx)

        sched = []
        for tab, idx, out in ((utab, uidx, out_u),
                              (itab, pidx, out_p),
                              (itab, nidx, out_n)):
            for c in range(NCH):
                sched.append((tab, idx, out, c * CH))
        total = len(sched)

        def fire(slot):
            tab, idx, _, ofs = sched[slot]
            buf = bufs[slot % NBUF]
            sems = gsems[slot % NBUF]

            def issue(g, carry):
                v = idx[pl.ds(ofs + g * 16, 16)]
                for l in range(16):
                    i = v[l]
                    pltpu.async_copy(tab.at[pl.ds(i, 1)],
                                     buf.at[pl.ds(g * 16 + l, 1)],
                                     sems[l % NSEM])
                return carry

            lax.fori_loop(0, G, issue, 0)

        def drain_gathers(slot):
            tab = sched[slot][0]
            buf = bufs[slot % NBUF]
            sems = gsems[slot % NBUF]
            per_sem = CH // NSEM

            def one(j, carry):
                for k in range(NSEM):
                    pltpu.make_async_copy(tab.at[pl.ds(0, 1)],
                                          buf.at[pl.ds(0, 1)], sems[k]).wait()
                return carry

            lax.fori_loop(0, per_sem, one, 0)

        def start_writeback(slot):
            _, _, out, ofs = sched[slot]
            buf = bufs[slot % NBUF]
            pltpu.async_copy(buf, out.at[pl.ds(base + ofs, CH)], wsems[slot % NBUF])

        def drain_writeback(slot):
            _, _, out, ofs = sched[slot]
            buf = bufs[slot % NBUF]
            pltpu.make_async_copy(buf, out.at[pl.ds(base + ofs, CH)],
                                  wsems[slot % NBUF]).wait()

        for s in range(min(NBUF - 1, total)):
            fire(s)
        for s in range(total):
            drain_gathers(s)
            start_writeback(s)
            nxt = s + NBUF - 1
            if nxt < total:
                prev = nxt - NBUF
                if prev >= 0:
                    drain_writeback(prev)
                fire(nxt)
        for s in range(total - NBUF, total):
            if s >= 0:
                drain_writeback(s)

    return sc_gather3, NW, b_per_w


def kernel(user, pos, neg, user_table, item_table):
    info = plsc.get_sparse_core_info()
    fn, nw, bw = _build(info.num_cores, info.num_subcores)
    u = user.astype(jnp.int32).reshape(nw, bw)
    p = pos.astype(jnp.int32).reshape(nw, bw)
    n = neg.astype(jnp.int32).reshape(nw, bw)
    return fn(u, p, n, user_table, item_table)
